# Initial kernel scaffold; baseline (speedup 1.0000x reference)
import jax
import jax.numpy as jnp
from jax.experimental import pallas as pl


def kernel(inputs, table):
    raise NotImplementedError("write your pallas kernel here")


# speedup vs baseline: 1.0938x; 1.0938x over previous
import functools

import jax
import jax.numpy as jnp
from jax import lax
from jax.experimental import pallas as pl
from jax.experimental.pallas import tpu as pltpu
from jax.experimental.pallas import tpu_sc as plsc

LANES = 128
CHUNK = 1024
GATHERS = CHUNK // LANES


@functools.lru_cache(maxsize=None)
def _make_lookup(vocab, batch_flat, dim):
    info = plsc.get_sparse_core_info()
    nc, ns = info.num_cores, info.num_subcores
    nw = nc * ns
    b_per_w = batch_flat // nw
    n_chunks = b_per_w // CHUNK
    idx_rows = CHUNK // LANES

    mesh = plsc.VectorSubcoreMesh(core_axis_name="c", subcore_axis_name="s")

    @functools.partial(
        pl.kernel,
        mesh=mesh,
        out_type=jax.ShapeDtypeStruct((batch_flat, dim), jnp.float32),
        scratch_types=[
            pltpu.VMEM((idx_rows, LANES), jnp.int32),
            pltpu.VMEM((CHUNK, dim), jnp.float32),
            pltpu.SemaphoreType.DMA,
        ],
        compiler_params=pltpu.CompilerParams(use_tc_tiling_on_sc=False),
    )
    def lookup(table_hbm, ids_hbm, out_hbm, idx_v, rows_v, sem):
        wid = lax.axis_index("s") * nc + lax.axis_index("c")
        row0 = wid * (b_per_w // LANES)

        def body(g, carry):
            base = wid * b_per_w + g * CHUNK
            pltpu.sync_copy(
                ids_hbm.at[pl.ds(row0 + g * idx_rows, idx_rows)], idx_v)
            descs = [
                pltpu.async_copy(
                    table_hbm.at[idx_v.at[j]],
                    rows_v.at[pl.ds(j * LANES, LANES)],
                    sem,
                )
                for j in range(GATHERS)
            ]
            for d in descs:
                d.wait()
            pltpu.sync_copy(rows_v, out_hbm.at[pl.ds(base, CHUNK)])
            return carry

        lax.fori_loop(0, n_chunks, body, 0)

    return lookup


def kernel(inputs, table):
    ids = inputs
    if ids.ndim > 2:
        ids = jnp.squeeze(ids, axis=-1)
    batch, hist = ids.shape
    vocab, dim = table.shape
    bf = batch * hist
    ids2d = ids.reshape(bf // LANES, LANES)
    out = _make_lookup(vocab, bf, dim)(table, ids2d)
    return out.reshape(batch, hist, dim)


# speedup vs baseline: 1.1127x; 1.0173x over previous
import functools

import jax
import jax.numpy as jnp
from jax import lax
from jax.experimental import pallas as pl
from jax.experimental.pallas import tpu as pltpu
from jax.experimental.pallas import tpu_sc as plsc

LANES = 128
CHUNK = 1280
GATHERS = CHUNK // LANES


@functools.lru_cache(maxsize=None)
def _make_lookup(vocab, batch_flat, dim):
    info = plsc.get_sparse_core_info()
    nc, ns = info.num_cores, info.num_subcores
    nw = nc * ns
    b_per_w = batch_flat // nw
    n_chunks = b_per_w // CHUNK
    n_groups = n_chunks // 2
    idx_rows = b_per_w // LANES

    mesh = plsc.VectorSubcoreMesh(core_axis_name="c", subcore_axis_name="s")

    @functools.partial(
        pl.kernel,
        mesh=mesh,
        out_type=jax.ShapeDtypeStruct((batch_flat, dim), jnp.float32),
        scratch_types=[
            pltpu.VMEM((idx_rows, LANES), jnp.int32),
            pltpu.VMEM((CHUNK, dim), jnp.float32),
            pltpu.VMEM((CHUNK, dim), jnp.float32),
            pltpu.SemaphoreType.DMA,
            pltpu.SemaphoreType.DMA,
            pltpu.SemaphoreType.DMA,
            pltpu.SemaphoreType.DMA,
        ],
        compiler_params=pltpu.CompilerParams(use_tc_tiling_on_sc=False),
    )
    def lookup(table_hbm, ids_hbm, out_hbm, idx_v, buf0, buf1,
               semg0, semg1, sems0, sems1):
        wid = lax.axis_index("s") * nc + lax.axis_index("c")
        pltpu.sync_copy(ids_hbm.at[pl.ds(wid * idx_rows, idx_rows)], idx_v)
        base = wid * b_per_w

        def fire_gathers(g, buf, semg):
            for j in range(GATHERS):
                pltpu.async_copy(
                    table_hbm.at[idx_v.at[g * GATHERS + j]],
                    buf.at[pl.ds(j * LANES, LANES)],
                    semg,
                )

        def wait_gathers(buf, semg):
            pltpu.make_async_copy(
                table_hbm.at[pl.ds(0, CHUNK)], buf, semg).wait()

        def fire_store(g, buf, sems):
            pltpu.async_copy(buf, out_hbm.at[pl.ds(base + g * CHUNK, CHUNK)],
                             sems)

        def wait_store(buf, sems):
            pltpu.make_async_copy(buf, out_hbm.at[pl.ds(base, CHUNK)],
                                  sems).wait()

        fire_gathers(0, buf0, semg0)

        def body(gg, carry):
            g0 = 2 * gg

            @pl.when(gg > 0)
            def _():
                wait_store(buf1, sems1)

            fire_gathers(g0 + 1, buf1, semg1)
            wait_gathers(buf0, semg0)
            fire_store(g0, buf0, sems0)

            @pl.when(gg < n_groups - 1)
            def _():
                wait_store(buf0, sems0)
                fire_gathers(g0 + 2, buf0, semg0)

            wait_gathers(buf1, semg1)
            fire_store(g0 + 1, buf1, sems1)
            return carry

        lax.fori_loop(0, n_groups, body, 0)
        wait_store(buf0, sems0)
        wait_store(buf1, sems1)

    return lookup


def kernel(inputs, table):
    ids = inputs
    if ids.ndim > 2:
        ids = jnp.squeeze(ids, axis=-1)
    batch, hist = ids.shape
    vocab, dim = table.shape
    bf = batch * hist
    ids2d = ids.reshape(bf // LANES, LANES)
    out = _make_lookup(vocab, bf, dim)(table, ids2d)
    return out.reshape(batch, hist, dim)


# speedup vs baseline: 1.5425x; 1.3862x over previous
import functools

import jax
import jax.numpy as jnp
from jax import lax
from jax.experimental import pallas as pl
from jax.experimental.pallas import tpu as pltpu
from jax.experimental.pallas import tpu_sc as plsc

LANES = 128
B_PER_W = 1024
SUB = 16


@functools.lru_cache(maxsize=None)
def _make_lookup(vocab, batch, hist, dim):
    info = plsc.get_sparse_core_info()
    nc, ns = info.num_cores, info.num_subcores
    h_half = hist // 2
    n_bw = batch // B_PER_W
    runs = B_PER_W // LANES
    n_chunks = h_half * runs
    sr = dim * 4

    mesh = plsc.VectorSubcoreMesh(core_axis_name="c", subcore_axis_name="s")

    @functools.partial(
        pl.kernel,
        mesh=mesh,
        out_type=jax.ShapeDtypeStruct((hist, dim, batch), jnp.float32),
        scratch_types=[
            pltpu.VMEM((h_half * B_PER_W,), jnp.int32),
            pltpu.VMEM((LANES,), jnp.int32),
            pltpu.VMEM((LANES,), jnp.int32),
            pltpu.VMEM((LANES, sr), jnp.float32),
            pltpu.VMEM((LANES, sr), jnp.float32),
            pltpu.VMEM((dim, LANES), jnp.float32),
            pltpu.VMEM((dim, LANES), jnp.float32),
            pltpu.SemaphoreType.DMA,
            pltpu.SemaphoreType.DMA,
            pltpu.SemaphoreType.DMA,
            pltpu.SemaphoreType.DMA,
        ],
        compiler_params=pltpu.CompilerParams(use_tc_tiling_on_sc=True,
                                             needs_layout_passes=False),
    )
    def lookup(table_hbm, ids_hbm, out_hbm, idx_v, sb0, sb1, ch0, ch1,
               tr0, tr1, semg0, semg1, semt0, semt1):
        wid = lax.axis_index("s") * nc + lax.axis_index("c")
        hh = wid // n_bw
        bb = wid % n_bw
        b0 = bb * B_PER_W

        for hl in range(h_half):
            pltpu.sync_copy(
                ids_hbm.at[pl.ds((hh * h_half + hl) * batch + b0, B_PER_W)],
                idx_v.at[pl.ds(hl * B_PER_W, B_PER_W)])

        def compute_sidx(g, sb):
            p0 = g * LANES
            for g8 in range(LANES // SUB):
                v = idx_v[pl.ds(p0 + g8 * SUB, SUB)]
                sb[pl.ds(g8 * SUB, SUB)] = v >> 2

        def fire_gather(sb, ch, semg):
            pltpu.async_copy(table_hbm.at[sb], ch, semg)

        def wait_gather(ch, semg):
            pltpu.make_async_copy(table_hbm.at[pl.ds(0, LANES)], ch,
                                  semg).wait()

        def extract(g, ch, tr):
            p0 = g * LANES
            ivec = lax.iota(jnp.int32, SUB)
            for g8 in range(LANES // SUB):
                v = idx_v[pl.ds(p0 + g8 * SUB, SUB)]
                jbase = (v & 3) * dim
                rows = ivec + g8 * SUB
                for d in range(dim):
                    vals = plsc.load_gather(ch, [rows, jbase + d])
                    tr[d, pl.ds(g8 * SUB, SUB)] = vals

        def fire_store(g, tr, semt):
            h = hh * h_half + g // runs
            bcol = b0 + (g % runs) * LANES
            pltpu.async_copy(tr, out_hbm.at[h, :, pl.ds(bcol, LANES)], semt)

        def wait_store(tr, semt):
            pltpu.make_async_copy(tr, out_hbm.at[0, :, pl.ds(0, LANES)],
                                  semt).wait()

        compute_sidx(0, sb0)
        fire_gather(sb0, ch0, semg0)

        def body(gg, carry):
            g0 = 2 * gg

            compute_sidx(g0 + 1, sb1)
            fire_gather(sb1, ch1, semg1)
            wait_gather(ch0, semg0)

            @pl.when(gg > 0)
            def _():
                wait_store(tr0, semt0)

            extract(g0, ch0, tr0)
            fire_store(g0, tr0, semt0)

            @pl.when(gg < n_chunks // 2 - 1)
            def _():
                compute_sidx(g0 + 2, sb0)
                fire_gather(sb0, ch0, semg0)

            wait_gather(ch1, semg1)

            @pl.when(gg > 0)
            def _():
                wait_store(tr1, semt1)

            extract(g0 + 1, ch1, tr1)
            fire_store(g0 + 1, tr1, semt1)
            return carry

        lax.fori_loop(0, n_chunks // 2, body, 0)
        wait_store(tr0, semt0)
        wait_store(tr1, semt1)

    return lookup


def kernel(inputs, table):
    ids = inputs
    if ids.ndim > 2:
        ids = jnp.squeeze(ids, axis=-1)
    batch, hist = ids.shape
    vocab, dim = table.shape
    ids_flat = jnp.transpose(ids).reshape(batch * hist)
    table_sr = table.reshape(vocab // 4, dim * 4)
    out_t = _make_lookup(vocab, batch, hist, dim)(table_sr, ids_flat)
    return jnp.transpose(out_t, (2, 0, 1))


# speedup vs baseline: 1.5455x; 1.0020x over previous
import functools

import jax
import jax.numpy as jnp
from jax import lax
from jax.experimental import pallas as pl
from jax.experimental.pallas import tpu as pltpu
from jax.experimental.pallas import tpu_sc as plsc

LANES = 128
CHUNK = 256
B_PER_W = 1024
SUB = 16


@functools.lru_cache(maxsize=None)
def _make_lookup(vocab, batch, hist, dim):
    info = plsc.get_sparse_core_info()
    nc, ns = info.num_cores, info.num_subcores
    h_half = hist // 2
    n_bw = batch // B_PER_W
    runs = B_PER_W // CHUNK
    n_chunks = h_half * runs
    per_w = h_half * B_PER_W
    sr = dim * 4
    streams = CHUNK // LANES

    mesh = plsc.VectorSubcoreMesh(core_axis_name="c", subcore_axis_name="s")

    @functools.partial(
        pl.kernel,
        mesh=mesh,
        out_type=jax.ShapeDtypeStruct((hist, dim, batch), jnp.float32),
        scratch_types=[
            pltpu.VMEM((per_w,), jnp.int32),
            pltpu.VMEM((streams, LANES), jnp.int32),
            pltpu.VMEM((streams, LANES), jnp.int32),
            pltpu.VMEM((CHUNK, sr), jnp.float32),
            pltpu.VMEM((CHUNK, sr), jnp.float32),
            pltpu.VMEM((dim, CHUNK), jnp.float32),
            pltpu.VMEM((dim, CHUNK), jnp.float32),
            pltpu.SemaphoreType.DMA,
            pltpu.SemaphoreType.DMA,
            pltpu.SemaphoreType.DMA,
            pltpu.SemaphoreType.DMA,
        ],
        compiler_params=pltpu.CompilerParams(use_tc_tiling_on_sc=True,
                                             needs_layout_passes=False),
    )
    def lookup(table_hbm, ids_hbm, out_hbm, idx_v, sb0, sb1, ch0, ch1,
               tr0, tr1, semg0, semg1, semt0, semt1):
        wid = lax.axis_index("s") * nc + lax.axis_index("c")
        hh = wid // n_bw
        bb = wid % n_bw
        b0 = bb * B_PER_W

        pltpu.sync_copy(ids_hbm.at[pl.ds(wid * per_w, per_w)], idx_v)

        def compute_sidx(g, sb):
            p0 = g * CHUNK
            for j in range(streams):
                for g8 in range(LANES // SUB):
                    v = idx_v[pl.ds(p0 + j * LANES + g8 * SUB, SUB)]
                    sb[j, pl.ds(g8 * SUB, SUB)] = v >> 2

        def fire_gather(sb, ch, semg):
            for j in range(streams):
                pltpu.async_copy(table_hbm.at[sb.at[j]],
                                 ch.at[pl.ds(j * LANES, LANES)], semg)

        def wait_gather(ch, semg):
            pltpu.make_async_copy(table_hbm.at[pl.ds(0, CHUNK)], ch,
                                  semg).wait()

        def extract(g, ch, tr):
            p0 = g * CHUNK
            ivec = lax.iota(jnp.int32, SUB)
            for g8 in range(CHUNK // SUB):
                v = idx_v[pl.ds(p0 + g8 * SUB, SUB)]
                jbase = (v & 3) * dim
                rows = ivec + g8 * SUB
                for d in range(dim):
                    vals = plsc.load_gather(ch, [rows, jbase + d])
                    tr[d, pl.ds(g8 * SUB, SUB)] = vals

        def fire_store(g, tr, semt):
            h = hh * h_half + g // runs
            bcol = b0 + (g % runs) * CHUNK
            pltpu.async_copy(tr, out_hbm.at[h, :, pl.ds(bcol, CHUNK)], semt)

        def wait_store(tr, semt):
            pltpu.make_async_copy(tr, out_hbm.at[0, :, pl.ds(0, CHUNK)],
                                  semt).wait()

        compute_sidx(0, sb0)
        fire_gather(sb0, ch0, semg0)

        def body(gg, carry):
            g0 = 2 * gg

            compute_sidx(g0 + 1, sb1)
            fire_gather(sb1, ch1, semg1)
            wait_gather(ch0, semg0)

            @pl.when(gg > 0)
            def _():
                wait_store(tr0, semt0)

            extract(g0, ch0, tr0)
            fire_store(g0, tr0, semt0)

            @pl.when(gg < n_chunks // 2 - 1)
            def _():
                compute_sidx(g0 + 2, sb0)
                fire_gather(sb0, ch0, semg0)

            wait_gather(ch1, semg1)

            @pl.when(gg > 0)
            def _():
                wait_store(tr1, semt1)

            extract(g0 + 1, ch1, tr1)
            fire_store(g0 + 1, tr1, semt1)
            return carry

        lax.fori_loop(0, n_chunks // 2, body, 0)
        wait_store(tr0, semt0)
        wait_store(tr1, semt1)

    return lookup


def kernel(inputs, table):
    ids = inputs
    if ids.ndim > 2:
        ids = jnp.squeeze(ids, axis=-1)
    batch, hist = ids.shape
    vocab, dim = table.shape
    h_half = hist // 2
    n_bw = batch // B_PER_W
    ids_w = (jnp.transpose(ids)
             .reshape(2, h_half, n_bw, B_PER_W)
             .transpose(0, 2, 1, 3)
             .reshape(batch * hist))
    table_sr = table.reshape(vocab // 4, dim * 4)
    out_t = _make_lookup(vocab, batch, hist, dim)(table_sr, ids_w)
    return jnp.transpose(out_t, (2, 0, 1))


# speedup vs baseline: 1.9546x; 1.2647x over previous
import functools

import jax
import jax.numpy as jnp
from jax import lax
from jax.experimental import pallas as pl
from jax.experimental.pallas import tpu as pltpu
from jax.experimental.pallas import tpu_sc as plsc

LANES = 128
CHUNK = 256
B_PER_W = 1024
SUB = 16


@functools.lru_cache(maxsize=None)
def _make_lookup(vocab, batch, hist, dim):
    info = plsc.get_sparse_core_info()
    nc, ns = info.num_cores, info.num_subcores
    h_half = hist // 2
    n_bw = batch // B_PER_W
    runs = B_PER_W // CHUNK
    n_chunks = h_half * runs
    per_w = h_half * B_PER_W
    sr = dim * 4
    streams = CHUNK // LANES

    mesh = plsc.VectorSubcoreMesh(core_axis_name="c", subcore_axis_name="s")

    @functools.partial(
        pl.kernel,
        mesh=mesh,
        out_type=jax.ShapeDtypeStruct((hist, dim, batch), jnp.float32),
        scratch_types=[
            pltpu.VMEM((per_w,), jnp.int32),
            pltpu.VMEM((streams, LANES), jnp.int32),
            pltpu.VMEM((streams, LANES), jnp.int32),
            pltpu.VMEM((CHUNK, sr), jnp.float32),
            pltpu.VMEM((CHUNK, sr), jnp.float32),
            pltpu.VMEM((dim, CHUNK), jnp.float32),
            pltpu.VMEM((dim, CHUNK), jnp.float32),
            pltpu.SemaphoreType.DMA,
            pltpu.SemaphoreType.DMA,
            pltpu.SemaphoreType.DMA,
            pltpu.SemaphoreType.DMA,
        ],
        compiler_params=pltpu.CompilerParams(use_tc_tiling_on_sc=True,
                                             needs_layout_passes=False),
    )
    def lookup(table_hbm, ids_hbm, out_hbm, idx_v, sb0, sb1, ch0, ch1,
               tr0, tr1, semg0, semg1, semt0, semt1):
        wid = lax.axis_index("s") * nc + lax.axis_index("c")
        hh = wid // n_bw
        bb = wid % n_bw
        b0 = bb * B_PER_W

        pltpu.sync_copy(ids_hbm.at[pl.ds(wid * per_w, per_w)], idx_v)

        def compute_sidx(g, sb):
            p0 = g * CHUNK
            for j in range(streams):
                for g8 in range(LANES // SUB):
                    v = idx_v[pl.ds(p0 + j * LANES + g8 * SUB, SUB)]
                    sb[j, pl.ds(g8 * SUB, SUB)] = v >> 2

        def fire_gather(sb, ch, semg):
            for j in range(streams):
                pltpu.async_copy(table_hbm.at[sb.at[j]],
                                 ch.at[pl.ds(j * LANES, LANES)], semg)

        def wait_gather(ch, semg):
            pltpu.make_async_copy(table_hbm.at[pl.ds(0, CHUNK)], ch,
                                  semg).wait()

        def extract(g, ch, tr):
            p0 = g * CHUNK
            ivec = lax.iota(jnp.int32, SUB)

            @plsc.parallel_loop(0, CHUNK // SUB, unroll=4)
            def _(g8):
                v = idx_v[pl.ds(p0 + g8 * SUB, SUB)]
                jbase = (v & 3) * dim
                rows = ivec + g8 * SUB
                for d in range(dim):
                    vals = plsc.load_gather(ch, [rows, jbase + d])
                    tr[d, pl.ds(g8 * SUB, SUB)] = vals

        def fire_store(g, tr, semt):
            h = hh * h_half + g // runs
            bcol = b0 + (g % runs) * CHUNK
            pltpu.async_copy(tr, out_hbm.at[h, :, pl.ds(bcol, CHUNK)], semt)

        def wait_store(tr, semt):
            pltpu.make_async_copy(tr, out_hbm.at[0, :, pl.ds(0, CHUNK)],
                                  semt).wait()

        compute_sidx(0, sb0)
        fire_gather(sb0, ch0, semg0)

        def body(gg, carry):
            g0 = 2 * gg

            compute_sidx(g0 + 1, sb1)
            fire_gather(sb1, ch1, semg1)
            wait_gather(ch0, semg0)

            @pl.when(gg > 0)
            def _():
                wait_store(tr0, semt0)

            extract(g0, ch0, tr0)
            fire_store(g0, tr0, semt0)

            @pl.when(gg < n_chunks // 2 - 1)
            def _():
                compute_sidx(g0 + 2, sb0)
                fire_gather(sb0, ch0, semg0)

            wait_gather(ch1, semg1)

            @pl.when(gg > 0)
            def _():
                wait_store(tr1, semt1)

            extract(g0 + 1, ch1, tr1)
            fire_store(g0 + 1, tr1, semt1)
            return carry

        lax.fori_loop(0, n_chunks // 2, body, 0)
        wait_store(tr0, semt0)
        wait_store(tr1, semt1)

    return lookup


def kernel(inputs, table):
    ids = inputs
    if ids.ndim > 2:
        ids = jnp.squeeze(ids, axis=-1)
    batch, hist = ids.shape
    vocab, dim = table.shape
    h_half = hist // 2
    n_bw = batch // B_PER_W
    ids_w = (jnp.transpose(ids)
             .reshape(2, h_half, n_bw, B_PER_W)
             .transpose(0, 2, 1, 3)
             .reshape(batch * hist))
    table_sr = table.reshape(vocab // 4, dim * 4)
    out_t = _make_lookup(vocab, batch, hist, dim)(table_sr, ids_w)
    return jnp.transpose(out_t, (2, 0, 1))
